# baseline (device time: 144686 ns/iter reference)
import os

import jax
import jax.numpy as jnp
from jax import lax
from jax.experimental import pallas as pl
from jax.experimental.pallas import tpu as pltpu

_VARIANT = os.environ.get("KERNEL_VARIANT", "full")

N_DEV = 4
M_LOC = 1024
K = 4096
KB = K // 2
N_GLOB = 8192
N_LOC = N_GLOB // N_DEV
CB = 512
NBLK = N_LOC // CB
W_SLOTS = 3

GELU_C = 0.7978845608028654

F8 = jnp.float8_e4m3fn


def _gelu(y):
    return 0.5 * y * (1.0 + jnp.tanh(GELU_C * (y + 0.044715 * y * y * y)))


def kernel(x, w_mat):
    def body(x_ref, w_hbm, out_hbm, w_buf, send_bf, send_f8, recv_bf,
             recv_f8, out_stage, w_sems, send_bf_sems, send_f8_sems,
             recv_bf_sems, recv_f8_sems, out_sems):
        my = lax.axis_index("i")

        barrier = pltpu.get_barrier_semaphore()
        for s in range(1, N_DEV):
            pl.semaphore_signal(
                barrier, inc=1,
                device_id=((my + s) % N_DEV,),
                device_id_type=pl.DeviceIdType.MESH,
            )
        pl.semaphore_wait(barrier, N_DEV - 1)

        order = [(s, j) for j in range(NBLK) for s in (1, 2, 3)]
        order += [(0, j) for j in range(NBLK)]
        steps = [(s, j, kb) for (s, j) in order for kb in range(2)]

        def w_load(step_idx):
            s, j, kb = steps[step_idx]
            t = (my + s) % N_DEV
            cb = t * NBLK + j
            slot = step_idx % W_SLOTS
            return pltpu.make_async_copy(
                w_hbm.at[pl.ds(kb * KB, KB), pl.ds(cb * CB, CB)],
                w_buf.at[slot],
                w_sems.at[slot],
            )

        loads = {}
        if _VARIANT != "comm":
            for i in range(W_SLOTS - 1):
                loads[i] = w_load(i)
                loads[i].start()

        out_inflight = [None, None]
        send_rdmas = []
        oslot = 0

        def store_out(value_f32, row_start, col_start):
            nonlocal oslot
            if out_inflight[oslot] is not None:
                out_inflight[oslot].wait()
            out_stage[oslot] = value_f32
            dma = pltpu.make_async_copy(
                out_stage.at[oslot],
                out_hbm.at[pl.ds(row_start, M_LOC), pl.ds(col_start, CB)],
                out_sems.at[oslot],
            )
            dma.start()
            out_inflight[oslot] = dma
            oslot = (oslot + 1) % 2

        def send_rdma(s, j, t):
            if s == 2:
                return pltpu.make_async_remote_copy(
                    src_ref=send_f8.at[j], dst_ref=recv_f8.at[j],
                    send_sem=send_f8_sems.at[j], recv_sem=recv_f8_sems.at[j],
                    device_id=(t,), device_id_type=pl.DeviceIdType.MESH,
                )
            g = 1 if s == 1 else 0
            sb = g * NBLK + j
            return pltpu.make_async_remote_copy(
                src_ref=send_bf.at[sb], dst_ref=recv_bf.at[sb],
                send_sem=send_bf_sems.at[sb], recv_sem=recv_bf_sems.at[sb],
                device_id=(t,), device_id_type=pl.DeviceIdType.MESH,
            )

        y_partial = None
        for step_idx, (s, j, kb) in enumerate(steps):
            t = (my + s) % N_DEV
            if _VARIANT == "comm":
                if kb == 0:
                    continue
                g = None
            else:
                nxt = step_idx + W_SLOTS - 1
                if nxt < len(steps):
                    loads[nxt] = w_load(nxt)
                    loads[nxt].start()
                loads[step_idx].wait()

                wblk = w_buf[step_idx % W_SLOTS].astype(jnp.bfloat16)
                yk = jnp.dot(x_ref[:, kb * KB:(kb + 1) * KB], wblk,
                             preferred_element_type=jnp.float32)
                if kb == 0:
                    y_partial = yk
                    continue
                g = _gelu(y_partial + yk)

            if _VARIANT == "compute":
                store_out(g, my * M_LOC, j * CB)
                continue

            if s != 0:
                if _VARIANT == "comm":
                    if s == 2:
                        send_f8[j] = x_ref[:, :CB].astype(F8)
                    else:
                        send_bf[(1 if s == 1 else 0) * NBLK + j] = \
                            x_ref[:, :CB]
                elif s == 2:
                    send_f8[j] = g.astype(F8)
                else:
                    send_bf[(1 if s == 1 else 0) * NBLK + j] = \
                        g.astype(jnp.bfloat16)
                rdma = send_rdma(s, j, t)
                rdma.start()
                send_rdmas.append(rdma)
            else:
                if _VARIANT != "comm":
                    store_out(g, my * M_LOC, j * CB)

        for j in range(NBLK) if _VARIANT != "compute" else ():
            for sp in (3, 2, 1):
                o = (my + sp) % N_DEV
                if sp == 2:
                    recv = send_rdma(2, j, my)
                    recv.wait_recv()
                    val = recv_f8[j].astype(jnp.float32)
                else:
                    g = 1 if sp == 3 else 0
                    recv = send_rdma(4 - sp, j, my)
                    recv.wait_recv()
                    val = recv_bf[g * NBLK + j].astype(jnp.float32)
                store_out(val, o * M_LOC, j * CB)

        for rdma in send_rdmas:
            rdma.wait_send()
        for dma in out_inflight:
            if dma is not None:
                dma.wait()

    out_shape = jax.ShapeDtypeStruct((N_DEV * M_LOC, N_LOC), jnp.float32)
    return pl.pallas_call(
        body,
        out_shape=out_shape,
        in_specs=[
            pl.BlockSpec(memory_space=pltpu.MemorySpace.VMEM),
            pl.BlockSpec(memory_space=pltpu.MemorySpace.HBM),
        ],
        out_specs=pl.BlockSpec(memory_space=pltpu.MemorySpace.HBM),
        scratch_shapes=[
            pltpu.VMEM((W_SLOTS, KB, CB), jnp.float32),
            pltpu.VMEM((2 * NBLK, M_LOC, CB), jnp.bfloat16),
            pltpu.VMEM((NBLK, M_LOC, CB), F8),
            pltpu.VMEM((2 * NBLK, M_LOC, CB), jnp.bfloat16),
            pltpu.VMEM((NBLK, M_LOC, CB), F8),
            pltpu.VMEM((2, M_LOC, CB), jnp.float32),
            pltpu.SemaphoreType.DMA((W_SLOTS,)),
            pltpu.SemaphoreType.DMA((2 * NBLK,)),
            pltpu.SemaphoreType.DMA((NBLK,)),
            pltpu.SemaphoreType.DMA((2 * NBLK,)),
            pltpu.SemaphoreType.DMA((NBLK,)),
            pltpu.SemaphoreType.DMA((2,)),
        ],
        compiler_params=pltpu.CompilerParams(
            collective_id=0,
            vmem_limit_bytes=56 * 1024 * 1024,
        ),
    )(x.astype(jnp.bfloat16), w_mat)


# device time: 135286 ns/iter; 1.0695x vs baseline; 1.0695x over previous
import os

import jax
import jax.numpy as jnp
from jax import lax
from jax.experimental import pallas as pl
from jax.experimental.pallas import tpu as pltpu

_VARIANT = os.environ.get("KERNEL_VARIANT", "full")

N_DEV = 4
M_LOC = 1024
K = 4096
N_GLOB = 8192
N_LOC = N_GLOB // N_DEV
CB = 512
NBLK = N_LOC // CB
W_SLOTS = 2

GELU_C = 0.7978845608028654

F8 = jnp.float8_e4m3fn


def _gelu(y):
    return 0.5 * y * (1.0 + jnp.tanh(GELU_C * (y + 0.044715 * y * y * y)))


def kernel(x, w_mat):
    def body(x_ref, w_hbm, out_hbm, w_buf, send_bf, send_f8, recv_bf,
             recv_f8, out_stage, w_sems, send_bf_sems, send_f8_sems,
             recv_bf_sems, recv_f8_sems, out_sems):
        my = lax.axis_index("i")

        barrier = pltpu.get_barrier_semaphore()
        for s in range(1, N_DEV):
            pl.semaphore_signal(
                barrier, inc=1,
                device_id=((my + s) % N_DEV,),
                device_id_type=pl.DeviceIdType.MESH,
            )
        pl.semaphore_wait(barrier, N_DEV - 1)

        steps = [(s, j) for j in range(NBLK) for s in (1, 2, 3)]
        steps += [(0, j) for j in range(NBLK)]

        def w_load(step_idx):
            s, j = steps[step_idx]
            t = (my + s) % N_DEV
            cb = t * NBLK + j
            slot = step_idx % W_SLOTS
            return pltpu.make_async_copy(
                w_hbm.at[:, pl.ds(cb * CB, CB)],
                w_buf.at[slot],
                w_sems.at[slot],
            )

        loads = {}
        if _VARIANT != "comm":
            for i in range(W_SLOTS - 1):
                loads[i] = w_load(i)
                loads[i].start()

        out_inflight = [None, None]
        send_rdmas = []
        oslot = 0

        def store_out(value_f32, row_start, col_start):
            nonlocal oslot
            if out_inflight[oslot] is not None:
                out_inflight[oslot].wait()
            out_stage[oslot] = value_f32
            dma = pltpu.make_async_copy(
                out_stage.at[oslot],
                out_hbm.at[pl.ds(row_start, M_LOC), pl.ds(col_start, CB)],
                out_sems.at[oslot],
            )
            dma.start()
            out_inflight[oslot] = dma
            oslot = (oslot + 1) % 2

        def send_rdma(s, j, t):
            if s == 2:
                return pltpu.make_async_remote_copy(
                    src_ref=send_f8.at[j], dst_ref=recv_f8.at[j],
                    send_sem=send_f8_sems.at[j], recv_sem=recv_f8_sems.at[j],
                    device_id=(t,), device_id_type=pl.DeviceIdType.MESH,
                )
            g = 1 if s == 1 else 0
            sb = g * NBLK + j
            return pltpu.make_async_remote_copy(
                src_ref=send_bf.at[sb], dst_ref=recv_bf.at[sb],
                send_sem=send_bf_sems.at[sb], recv_sem=recv_bf_sems.at[sb],
                device_id=(t,), device_id_type=pl.DeviceIdType.MESH,
            )

        if _VARIANT == "hbm":
            for step_idx in range(len(steps)):
                nxt = step_idx + W_SLOTS - 1
                if nxt < len(steps):
                    loads[nxt] = w_load(nxt)
                    loads[nxt].start()
                loads[step_idx].wait()
            store_out(w_buf[0][:M_LOC].astype(jnp.float32), 0, 0)
            out_inflight[0].wait()
            return

        for step_idx, (s, j) in enumerate(steps):
            t = (my + s) % N_DEV
            if _VARIANT == "comm":
                g = None
            else:
                nxt = step_idx + W_SLOTS - 1
                if nxt < len(steps):
                    loads[nxt] = w_load(nxt)
                    loads[nxt].start()
                loads[step_idx].wait()

                wblk = w_buf[step_idx % W_SLOTS].astype(jnp.bfloat16)
                y = jnp.dot(x_ref[...], wblk,
                            preferred_element_type=jnp.float32)
                g = _gelu(y)

            if _VARIANT == "compute":
                store_out(g, my * M_LOC, j * CB)
                continue

            if s != 0:
                if _VARIANT == "comm":
                    if s == 2:
                        send_f8[j] = x_ref[:, :CB].astype(F8)
                    else:
                        send_bf[(1 if s == 1 else 0) * NBLK + j] = \
                            x_ref[:, :CB]
                elif s == 2:
                    send_f8[j] = g.astype(F8)
                else:
                    send_bf[(1 if s == 1 else 0) * NBLK + j] = \
                        g.astype(jnp.bfloat16)
                rdma = send_rdma(s, j, t)
                rdma.start()
                send_rdmas.append(rdma)
            else:
                if _VARIANT != "comm":
                    store_out(g, my * M_LOC, j * CB)

        for j in range(NBLK) if _VARIANT != "compute" else ():
            for sp in (3, 2, 1):
                o = (my + sp) % N_DEV
                if sp == 2:
                    recv = send_rdma(2, j, my)
                    recv.wait_recv()
                    val = recv_f8[j].astype(jnp.float32)
                else:
                    g = 1 if sp == 3 else 0
                    recv = send_rdma(4 - sp, j, my)
                    recv.wait_recv()
                    val = recv_bf[g * NBLK + j].astype(jnp.float32)
                store_out(val, o * M_LOC, j * CB)

        for rdma in send_rdmas:
            rdma.wait_send()
        for dma in out_inflight:
            if dma is not None:
                dma.wait()

    out_shape = jax.ShapeDtypeStruct((N_DEV * M_LOC, N_LOC), jnp.float32)
    return pl.pallas_call(
        body,
        out_shape=out_shape,
        in_specs=[
            pl.BlockSpec(memory_space=pltpu.MemorySpace.VMEM),
            pl.BlockSpec(memory_space=pltpu.MemorySpace.HBM),
        ],
        out_specs=pl.BlockSpec(memory_space=pltpu.MemorySpace.HBM),
        scratch_shapes=[
            pltpu.VMEM((W_SLOTS, K, CB), jnp.float32),
            pltpu.VMEM((2 * NBLK, M_LOC, CB), jnp.bfloat16),
            pltpu.VMEM((NBLK, M_LOC, CB), F8),
            pltpu.VMEM((2 * NBLK, M_LOC, CB), jnp.bfloat16),
            pltpu.VMEM((NBLK, M_LOC, CB), F8),
            pltpu.VMEM((2, M_LOC, CB), jnp.float32),
            pltpu.SemaphoreType.DMA((W_SLOTS,)),
            pltpu.SemaphoreType.DMA((2 * NBLK,)),
            pltpu.SemaphoreType.DMA((NBLK,)),
            pltpu.SemaphoreType.DMA((2 * NBLK,)),
            pltpu.SemaphoreType.DMA((NBLK,)),
            pltpu.SemaphoreType.DMA((2,)),
        ],
        compiler_params=pltpu.CompilerParams(
            collective_id=0,
            vmem_limit_bytes=56 * 1024 * 1024,
        ),
    )(x.astype(jnp.bfloat16), w_mat)
